# baseline (speedup 1.0000x reference)
import functools
import math

import jax
import jax.numpy as jnp
from jax import lax
from jax.experimental import pallas as pl
from jax.experimental.pallas import tpu as pltpu
from jax.experimental.pallas import tpu_sc as plsc

D_MODEL_K = 1024
VOCAB_K = 100000
SCALE_K = math.sqrt(D_MODEL_K)

_info = plsc.get_sparse_core_info()
_NC, _NS, _L = _info.num_cores, _info.num_subcores, _info.num_lanes
_NW = _NC * _NS


def _make_lookup(B: int, D: int):
    assert B % (8 * _NW) == 0 and D % _L == 0
    b_per_w = B // _NW
    CHUNK = 40
    sizes = [CHUNK] * (b_per_w // CHUNK)
    if b_per_w % CHUNK:
        sizes.append(b_per_w % CHUNK)
    assert all(s % 8 == 0 for s in sizes)
    offs = [sum(sizes[:k]) for k in range(len(sizes))]
    n_chunks = len(sizes)
    mesh = plsc.VectorSubcoreMesh(core_axis_name="c", subcore_axis_name="s")

    @functools.partial(
        pl.kernel,
        mesh=mesh,
        out_type=jax.ShapeDtypeStruct((B, D), jnp.float32),
        scratch_types=[
            pltpu.VMEM((b_per_w,), jnp.int32),
            pltpu.VMEM((CHUNK, D), jnp.float32),
            pltpu.VMEM((CHUNK, D), jnp.float32),
            pltpu.VMEM((CHUNK, D), jnp.float32),
            pltpu.SemaphoreType.DMA,
            pltpu.SemaphoreType.DMA,
            pltpu.SemaphoreType.DMA,
            pltpu.SemaphoreType.DMA,
            pltpu.SemaphoreType.DMA,
            pltpu.SemaphoreType.DMA,
        ],
    )
    def lookup(
        x_hbm, lut_hbm, out_hbm, idx_v, rows0, rows1, rows2, g0, g1, g2, s0, s1, s2
    ):
        wid = lax.axis_index("s") * _NC + lax.axis_index("c")
        base = wid * b_per_w
        pltpu.sync_copy(x_hbm.at[pl.ds(base, b_per_w)], idx_v)

        NBUF = 3
        bufs = (rows0, rows1, rows2)
        gsems = (g0, g1, g2)
        ssems = (s0, s1, s2)

        def gather(c):
            return pltpu.async_copy(
                lut_hbm.at[idx_v.at[pl.ds(offs[c], sizes[c])]],
                bufs[c % NBUF].at[pl.ds(0, sizes[c])],
                gsems[c % NBUF],
            )

        def scale(buf, rows):
            def scale_rows(i, carry):
                for r in range(2):
                    for j in range(D // _L):
                        sl = buf[2 * i + r, pl.ds(j * _L, _L)]
                        buf[2 * i + r, pl.ds(j * _L, _L)] = sl * SCALE_K
                return carry

            lax.fori_loop(0, rows // 2, scale_rows, 0)

        def store(c):
            return pltpu.async_copy(
                bufs[c % NBUF].at[pl.ds(0, sizes[c])],
                out_hbm.at[pl.ds(base + offs[c], sizes[c])],
                ssems[c % NBUF],
            )

        gathers = [gather(0)]
        stores = []
        for c in range(n_chunks):
            if c + 1 < n_chunks:
                if c >= NBUF - 1:
                    stores[c + 1 - NBUF].wait()
                gathers.append(gather(c + 1))
            gathers[c].wait()
            scale(bufs[c % NBUF], sizes[c])
            stores.append(store(c))
        for c in range(n_chunks - NBUF, n_chunks):
            stores[c].wait()

    return lookup


def kernel(x, lut):
    B = x.shape[0] * x.shape[1]
    D = lut.shape[1]
    flat_idx = jnp.reshape(x, (B,)).astype(jnp.int32)
    out = _make_lookup(B, D)(flat_idx, lut)
    return jnp.reshape(out, (*x.shape, D))


# speedup vs baseline: 1.1257x; 1.1257x over previous
import functools
import math

import jax
import jax.numpy as jnp
from jax import lax
from jax.experimental import pallas as pl
from jax.experimental.pallas import tpu as pltpu
from jax.experimental.pallas import tpu_sc as plsc

D_MODEL_K = 1024
VOCAB_K = 100000
SCALE_K = math.sqrt(D_MODEL_K)

_info = plsc.get_sparse_core_info()
_NC, _NS, _L = _info.num_cores, _info.num_subcores, _info.num_lanes
_NW = _NC * _NS


def _make_lookup(B: int, D: int):
    assert B % (8 * _NW) == 0 and D % _L == 0
    b_per_w = B // _NW
    CHUNK = 40
    sizes = [CHUNK] * (b_per_w // CHUNK)
    if b_per_w % CHUNK:
        sizes.append(b_per_w % CHUNK)
    assert all(s % 8 == 0 for s in sizes)
    offs = [sum(sizes[:k]) for k in range(len(sizes))]
    n_chunks = len(sizes)
    mesh = plsc.VectorSubcoreMesh(core_axis_name="c", subcore_axis_name="s")

    @functools.partial(
        pl.kernel,
        mesh=mesh,
        out_type=jax.ShapeDtypeStruct((B, D), jnp.float32),
        scratch_types=[
            pltpu.VMEM((b_per_w,), jnp.int32),
            pltpu.VMEM((CHUNK, D), jnp.float32),
            pltpu.VMEM((CHUNK, D), jnp.float32),
            pltpu.VMEM((CHUNK, D), jnp.float32),
            pltpu.SemaphoreType.DMA,
            pltpu.SemaphoreType.DMA,
            pltpu.SemaphoreType.DMA,
            pltpu.SemaphoreType.DMA,
            pltpu.SemaphoreType.DMA,
            pltpu.SemaphoreType.DMA,
        ],
    )
    def lookup(
        x_hbm, lut_hbm, out_hbm, idx_v, rows0, rows1, rows2, g0, g1, g2, s0, s1, s2
    ):
        wid = lax.axis_index("s") * _NC + lax.axis_index("c")
        base = wid * b_per_w
        pltpu.sync_copy(x_hbm.at[pl.ds(base, b_per_w)], idx_v)

        NBUF = 3
        bufs = (rows0, rows1, rows2)
        gsems = (g0, g1, g2)
        ssems = (s0, s1, s2)

        def gather(c):
            return pltpu.async_copy(
                lut_hbm.at[idx_v.at[pl.ds(offs[c], sizes[c])]],
                bufs[c % NBUF].at[pl.ds(0, sizes[c])],
                gsems[c % NBUF],
            )

        def scale(buf, rows):
            @plsc.parallel_loop(0, rows, 1, unroll=1)
            def scale_row(i):
                for j in range(D // _L):
                    sl = buf[i, pl.ds(j * _L, _L)]
                    buf[i, pl.ds(j * _L, _L)] = sl * SCALE_K

        def store(c):
            return pltpu.async_copy(
                bufs[c % NBUF].at[pl.ds(0, sizes[c])],
                out_hbm.at[pl.ds(base + offs[c], sizes[c])],
                ssems[c % NBUF],
            )

        gathers = [gather(0)]
        stores = []
        for c in range(n_chunks):
            if c + 1 < n_chunks:
                if c >= NBUF - 1:
                    stores[c + 1 - NBUF].wait()
                gathers.append(gather(c + 1))
            gathers[c].wait()
            scale(bufs[c % NBUF], sizes[c])
            stores.append(store(c))
        for c in range(n_chunks - NBUF, n_chunks):
            stores[c].wait()

    return lookup


def kernel(x, lut):
    B = x.shape[0] * x.shape[1]
    D = lut.shape[1]
    flat_idx = jnp.reshape(x, (B,)).astype(jnp.int32)
    out = _make_lookup(B, D)(flat_idx, lut)
    return jnp.reshape(out, (*x.shape, D))


# speedup vs baseline: 1.1490x; 1.0207x over previous
import functools
import math

import jax
import jax.numpy as jnp
from jax import lax
from jax.experimental import pallas as pl
from jax.experimental.pallas import tpu as pltpu
from jax.experimental.pallas import tpu_sc as plsc

D_MODEL_K = 1024
VOCAB_K = 100000
SCALE_K = math.sqrt(D_MODEL_K)

_info = plsc.get_sparse_core_info()
_NC, _NS, _L = _info.num_cores, _info.num_subcores, _info.num_lanes
_NW = _NC * _NS


def _make_lookup(B: int, D: int):
    assert B % (8 * _NW) == 0 and D % _L == 0
    b_per_w = B // _NW
    CHUNK = 40
    sizes = [CHUNK] * (b_per_w // CHUNK)
    if b_per_w % CHUNK:
        sizes.append(b_per_w % CHUNK)
    assert all(s % 8 == 0 for s in sizes)
    offs = [sum(sizes[:k]) for k in range(len(sizes))]
    n_chunks = len(sizes)
    mesh = plsc.VectorSubcoreMesh(core_axis_name="c", subcore_axis_name="s")

    @functools.partial(
        pl.kernel,
        mesh=mesh,
        out_type=jax.ShapeDtypeStruct((B, D), jnp.float32),
        scratch_types=[
            pltpu.VMEM((b_per_w,), jnp.int32),
            pltpu.VMEM((CHUNK, D), jnp.float32),
            pltpu.VMEM((CHUNK, D), jnp.float32),
            pltpu.VMEM((CHUNK, D), jnp.float32),
            pltpu.SemaphoreType.DMA,
            pltpu.SemaphoreType.DMA,
            pltpu.SemaphoreType.DMA,
            pltpu.SemaphoreType.DMA,
            pltpu.SemaphoreType.DMA,
            pltpu.SemaphoreType.DMA,
        ],
    )
    def lookup(
        x_hbm, lut_hbm, out_hbm, idx_v, rows0, rows1, rows2, g0, g1, g2, s0, s1, s2
    ):
        wid = lax.axis_index("s") * _NC + lax.axis_index("c")
        base = wid * b_per_w
        pltpu.sync_copy(x_hbm.at[pl.ds(base, b_per_w)], idx_v)

        NBUF = 3
        bufs = (rows0, rows1, rows2)
        gsems = (g0, g1, g2)
        ssems = (s0, s1, s2)

        def gather(c):
            return pltpu.async_copy(
                lut_hbm.at[idx_v.at[pl.ds(offs[c], sizes[c])]],
                bufs[c % NBUF].at[pl.ds(0, sizes[c])],
                gsems[c % NBUF],
            )

        def scale(buf, rows):
            def scale_row(i, carry):
                for j in range(D // _L):
                    sl = buf[i, pl.ds(j * _L, _L)]
                    buf[i, pl.ds(j * _L, _L)] = sl * SCALE_K
                return carry

            lax.fori_loop(0, rows, scale_row, 0)

        def store(c):
            return pltpu.async_copy(
                bufs[c % NBUF].at[pl.ds(0, sizes[c])],
                out_hbm.at[pl.ds(base + offs[c], sizes[c])],
                ssems[c % NBUF],
            )

        gathers = [gather(0)]
        stores = []
        for c in range(n_chunks):
            if c + 1 < n_chunks:
                if c >= NBUF - 1:
                    stores[c + 1 - NBUF].wait()
                gathers.append(gather(c + 1))
            gathers[c].wait()
            scale(bufs[c % NBUF], sizes[c])
            stores.append(store(c))
        for c in range(n_chunks - NBUF, n_chunks):
            stores[c].wait()

    return lookup


def kernel(x, lut):
    B = x.shape[0] * x.shape[1]
    D = lut.shape[1]
    flat_idx = jnp.reshape(x, (B,)).astype(jnp.int32)
    out = _make_lookup(B, D)(flat_idx, lut)
    return jnp.reshape(out, (*x.shape, D))
